# baseline (speedup 1.0000x reference)
import functools

import jax
import jax.numpy as jnp
from jax import lax
from jax.experimental import pallas as pl
from jax.experimental.pallas import tpu as pltpu
from jax.experimental.pallas import tpu_sc as plsc


def _sc_gather(table, idx):
    V, D = table.shape
    B = idx.shape[0]
    info = plsc.get_sparse_core_info()
    num_workers = info.num_cores * info.num_subcores
    b_per_w = B // num_workers
    mesh = plsc.VectorSubcoreMesh(core_axis_name="c", subcore_axis_name="s")

    @functools.partial(
        pl.kernel,
        mesh=mesh,
        out_type=jax.ShapeDtypeStruct((B, D), jnp.float32),
        scratch_types=[
            pltpu.VMEM((b_per_w,), jnp.int32),
            pltpu.VMEM((b_per_w, D), jnp.float32),
            pltpu.SemaphoreType.DMA,
        ],
    )
    def gather_kernel(table_hbm, idx_hbm, out_hbm, idx_v, rows_v, sem):
        wid = lax.axis_index("s") * info.num_cores + lax.axis_index("c")
        base = wid * b_per_w
        pltpu.sync_copy(idx_hbm.at[pl.ds(base, b_per_w)], idx_v)
        pltpu.async_copy(table_hbm.at[idx_v], rows_v, sem).wait()
        pltpu.sync_copy(rows_v, out_hbm.at[pl.ds(base, b_per_w)])

    return gather_kernel(table, idx)


def _sc_overall(w_flat, lat_b, C, N, NK, n_chunks):
    info = plsc.get_sparse_core_info()
    num_workers = info.num_cores * info.num_subcores
    rpw = C // num_workers
    mesh = plsc.VectorSubcoreMesh(core_axis_name="c", subcore_axis_name="s")

    @functools.partial(
        pl.kernel,
        mesh=mesh,
        out_type=jax.ShapeDtypeStruct((C, N), jnp.float32),
        scratch_types=[
            pltpu.VMEM((rpw, 128), jnp.float32),
            pltpu.VMEM((NK,), jnp.float32),
            pltpu.VMEM((rpw, NK), jnp.float32),
        ],
    )
    def overall_kernel(w_hbm, lat_hbm, out_hbm, latv, wbuf, obuf):
        wid = lax.axis_index("s") * info.num_cores + lax.axis_index("c")
        r0 = wid * rpw
        pltpu.sync_copy(lat_hbm.at[pl.ds(r0, rpw)], latv)
        lats = [latv[i, pl.ds(0, 16)] for i in range(rpw)]

        def chunk_body(k, carry):
            off = k * NK
            pltpu.sync_copy(w_hbm.at[pl.ds(off, NK)], wbuf)
            for j in range(NK // 16):
                wv = wbuf[pl.ds(16 * j, 16)]
                for i in range(rpw):
                    obuf[i, pl.ds(16 * j, 16)] = wv * lats[i]
            pltpu.sync_copy(obuf, out_hbm.at[pl.ds(r0, rpw), pl.ds(off, NK)])
            return carry

        lax.fori_loop(0, n_chunks, chunk_body, 0)

    return overall_kernel(w_flat, lat_b)


def _tail_body(alias_ref, lat_ref, w_ref, o_ref):
    o_ref[...] = lat_ref[...] * w_ref[...]


def _select_body(wp_ref, par_ref, o_ref):
    half = o_ref.shape[1]
    par = par_ref[...]
    o_ref[...] = jnp.where(par != 0, wp_ref[:, half:], wp_ref[:, :half])


def _overall_body(lat_ref, wov_ref, o_ref):
    o_ref[...] = lat_ref[...] * wov_ref[...]


def _height_body(lat_ref, wgt_ref, o_ref):
    o_ref[...] = lat_ref[...] * wgt_ref[...]


def kernel(latent, genes_oi, W_height, W_overall):
    C = latent.shape[0]
    G = genes_oi.shape[0]
    V, D = W_height.shape
    N = W_overall.shape[0]

    gi = genes_oi.astype(jnp.int32)
    packed = W_height.reshape(V // 2, 2 * D)
    wp = _sc_gather(packed, gi >> 1)

    wgt = pl.pallas_call(
        _select_body,
        out_shape=jax.ShapeDtypeStruct((G, D), jnp.float32),
    )(wp, (gi & 1).reshape(G, 1))

    NB = 3200
    out2 = pl.pallas_call(
        _overall_body,
        grid=(pl.cdiv(N, NB),),
        in_specs=[
            pl.BlockSpec((C, 1), lambda i: (0, 0)),
            pl.BlockSpec((1, NB), lambda i: (0, i)),
        ],
        out_specs=pl.BlockSpec((C, NB), lambda i: (0, i)),
        out_shape=jax.ShapeDtypeStruct((C, N), jnp.float32),
    )(latent.reshape(C, 1), W_overall.reshape(1, N))

    CB = 8
    GD = G * D
    out1 = pl.pallas_call(
        _height_body,
        grid=(C // CB,),
        in_specs=[
            pl.BlockSpec((CB, 1), lambda i: (i, 0)),
            pl.BlockSpec((1, GD), lambda i: (0, 0)),
        ],
        out_specs=pl.BlockSpec((CB, GD), lambda i: (i, 0)),
        out_shape=jax.ShapeDtypeStruct((C, GD), jnp.float32),
    )(latent.reshape(C, 1), wgt.reshape(1, GD))

    delta_height = out1.reshape(C, G, D)
    delta_overall = out2.reshape(C, N, 1)
    return delta_height, delta_overall


# speedup vs baseline: 1.8908x; 1.8908x over previous
import functools

import jax
import jax.numpy as jnp
from jax import lax
from jax.experimental import pallas as pl
from jax.experimental.pallas import tpu as pltpu
from jax.experimental.pallas import tpu_sc as plsc


def _sc_gather(table, idx):
    V, D = table.shape
    B = idx.shape[0]
    info = plsc.get_sparse_core_info()
    num_workers = info.num_cores * info.num_subcores
    b_per_w = B // num_workers
    mesh = plsc.VectorSubcoreMesh(core_axis_name="c", subcore_axis_name="s")

    @functools.partial(
        pl.kernel,
        mesh=mesh,
        out_type=jax.ShapeDtypeStruct((B, D), jnp.float32),
        scratch_types=[
            pltpu.VMEM((b_per_w,), jnp.int32),
            pltpu.VMEM((b_per_w, D), jnp.float32),
            pltpu.SemaphoreType.DMA,
        ],
    )
    def gather_kernel(table_hbm, idx_hbm, out_hbm, idx_v, rows_v, sem):
        wid = lax.axis_index("s") * info.num_cores + lax.axis_index("c")
        base = wid * b_per_w
        pltpu.sync_copy(idx_hbm.at[pl.ds(base, b_per_w)], idx_v)
        pltpu.async_copy(table_hbm.at[idx_v], rows_v, sem).wait()
        pltpu.sync_copy(rows_v, out_hbm.at[pl.ds(base, b_per_w)])

    return gather_kernel(table, idx)


def _sc_overall(w_flat, lat_b, C, N, NK, n_chunks):
    info = plsc.get_sparse_core_info()
    num_workers = info.num_cores * info.num_subcores
    rpw = C // num_workers
    mesh = plsc.VectorSubcoreMesh(core_axis_name="c", subcore_axis_name="s")

    @functools.partial(
        pl.kernel,
        mesh=mesh,
        out_type=jax.ShapeDtypeStruct((C, N), jnp.float32),
        scratch_types=[
            pltpu.VMEM((rpw, 128), jnp.float32),
            pltpu.VMEM((NK,), jnp.float32),
            pltpu.VMEM((rpw, NK), jnp.float32),
        ],
    )
    def overall_kernel(w_hbm, lat_hbm, out_hbm, latv, wbuf, obuf):
        wid = lax.axis_index("s") * info.num_cores + lax.axis_index("c")
        r0 = wid * rpw
        pltpu.sync_copy(lat_hbm.at[pl.ds(r0, rpw)], latv)
        lats = [latv[i, pl.ds(0, 16)] for i in range(rpw)]

        def chunk_body(k, carry):
            off = k * NK
            pltpu.sync_copy(w_hbm.at[pl.ds(off, NK)], wbuf)
            for j in range(NK // 16):
                wv = wbuf[pl.ds(16 * j, 16)]
                for i in range(rpw):
                    obuf[i, pl.ds(16 * j, 16)] = wv * lats[i]
            pltpu.sync_copy(obuf, out_hbm.at[pl.ds(r0, rpw), pl.ds(off, NK)])
            return carry

        lax.fori_loop(0, n_chunks, chunk_body, 0)

    return overall_kernel(w_flat, lat_b)


def _tail_body(alias_ref, lat_ref, w_ref, o_ref):
    o_ref[...] = lat_ref[...] * w_ref[...]


def _select_t_body(wp_ref, par_ref, o_ref):
    half = o_ref.shape[0]
    par = par_ref[...]
    sel = jnp.where(par != 0, wp_ref[:, half:], wp_ref[:, :half])
    o_ref[...] = sel.T


def _overall_body(lat_ref, wov_ref, o_ref):
    o_ref[...] = lat_ref[...] * wov_ref[...]


def _height_body(lat_ref, wgt_ref, o_ref):
    o_ref[...] = lat_ref[...] * wgt_ref[...]


def kernel(latent, genes_oi, W_height, W_overall):
    C = latent.shape[0]
    G = genes_oi.shape[0]
    V, D = W_height.shape
    N = W_overall.shape[0]

    gi = genes_oi.astype(jnp.int32)
    packed = W_height.reshape(V // 2, 2 * D)
    wp = _sc_gather(packed, gi >> 1)

    wgt = pl.pallas_call(
        _select_t_body,
        out_shape=jax.ShapeDtypeStruct((D, G), jnp.float32),
    )(wp, (gi & 1).reshape(G, 1))

    NB = 3200
    out2 = pl.pallas_call(
        _overall_body,
        grid=(pl.cdiv(N, NB),),
        in_specs=[
            pl.BlockSpec((C, 1), lambda i: (0, 0)),
            pl.BlockSpec((1, NB), lambda i: (0, i)),
        ],
        out_specs=pl.BlockSpec((C, NB), lambda i: (0, i)),
        out_shape=jax.ShapeDtypeStruct((C, N), jnp.float32),
    )(latent.reshape(C, 1), W_overall.reshape(1, N))

    CB = 8
    out1 = pl.pallas_call(
        _height_body,
        grid=(C // CB,),
        in_specs=[
            pl.BlockSpec((CB, 1, 1), lambda i: (i, 0, 0)),
            pl.BlockSpec((1, D, G), lambda i: (0, 0, 0)),
        ],
        out_specs=pl.BlockSpec((CB, D, G), lambda i: (i, 0, 0)),
        out_shape=jax.ShapeDtypeStruct((C, D, G), jnp.float32),
    )(latent.reshape(C, 1, 1), wgt.reshape(1, D, G))

    delta_height = out1.transpose(0, 2, 1)
    delta_overall = out2.reshape(C, N, 1)
    return delta_height, delta_overall
